# baseline (speedup 1.0000x reference)
import functools

import jax
import jax.numpy as jnp
from jax import lax
from jax.experimental import pallas as pl
from jax.experimental.pallas import tpu as pltpu
from jax.experimental.pallas import tpu_sc as plsc

NUM_NODES = 10000
NUM_EDGES = 320000
FEAT = 128

NC = 2
NS = 16
HALF = NUM_NODES // NC
ACC_ROWS = HALF + 8

CHUNK = 200
NCH_TOT = NUM_EDGES // CHUNK
SUB = 100
KSUB = CHUNK // SUB
NBUF = 3

ROWS_A = 312
ROWS_LAST = HALF - ROWS_A * (NS - 1)
ZROWS_A = 312
ZROWS_LAST = ACC_ROWS - ZROWS_A * (NS - 1)

_mesh = plsc.VectorSubcoreMesh(core_axis_name="c", subcore_axis_name="s")


def _split(count):
    out = [CHUNK] * (count // CHUNK)
    if count % CHUNK:
        out.append(count % CHUNK)
    return out


@functools.partial(
    pl.kernel,
    out_type=jax.ShapeDtypeStruct((NUM_NODES, FEAT), jnp.float32),
    mesh=_mesh,
    scratch_types=[
        pltpu.VMEM_SHARED((ACC_ROWS, FEAT), jnp.float32),
        pltpu.VMEM((NBUF, CHUNK, FEAT), jnp.float32),
        pltpu.VMEM((NBUF, KSUB, SUB), jnp.int32),
        pltpu.VMEM((NC, 16), jnp.float32),
    ] + [pltpu.SemaphoreType.DMA] * (2 * NBUF + 1),
    compiler_params=pltpu.CompilerParams(needs_layout_passes=False),
)
def _scatter_add(src_hbm, idx_hbm, bounds_hbm, out_hbm, acc, rows_v, idx_v,
                 bounds_v, *sems):
    rsems = sems[:NBUF]
    isems = sems[NBUF:2 * NBUF]
    ssem = sems[2 * NBUF]

    cid = lax.axis_index("c")
    sid = lax.axis_index("s")

    @pl.loop(0, CHUNK * FEAT // 16)
    def _(i):
        r = i // (FEAT // 16)
        col = i % (FEAT // 16)
        rows_v.at[0][r, pl.ds(col * 16, 16)] = jnp.zeros((16,), jnp.float32)

    def _owned(plan):
        a, last = plan
        yield (sid < NS - 1, sid * a, _split(a))
        yield (sid == NS - 1, (NS - 1) * a, _split(last))

    for cond, zbase, pieces in _owned((ZROWS_A, ZROWS_LAST)):
        @pl.when(cond)
        def _(zbase=zbase, pieces=pieces):
            off = 0
            for n in pieces:
                pltpu.sync_copy(rows_v.at[0].at[pl.ds(0, n)],
                                acc.at[pl.ds(zbase + off, n)])
                off += n
    plsc.subcore_barrier()

    pltpu.sync_copy(bounds_hbm, bounds_v)
    lo_end = jnp.sum(bounds_v[0, :]).astype(jnp.int32)
    hi_start = jnp.sum(bounds_v[1, :]).astype(jnp.int32)
    start_c = jnp.where(cid == 0, 0, hi_start)
    end_c = jnp.where(cid == 0, lo_end, NCH_TOT)
    trips = jnp.maximum(0, (end_c - start_c - sid + NS - 1) // NS)

    def chunk_of(t):
        return start_c + sid + t * NS

    def fill_start(t, b):
        ch = chunk_of(t)
        pltpu.async_copy(src_hbm.at[pl.ds(ch * CHUNK, CHUNK)],
                         rows_v.at[b], rsems[b])
        pltpu.async_copy(idx_hbm.at[ch], idx_v.at[b], isems[b])

    def fill_wait(t, b):
        ch = chunk_of(t)
        pltpu.make_async_copy(src_hbm.at[pl.ds(ch * CHUNK, CHUNK)],
                              rows_v.at[b], rsems[b]).wait()
        pltpu.make_async_copy(idx_hbm.at[ch], idx_v.at[b],
                              isems[b]).wait()

    rel_base = cid * HALF

    def clip(b):
        for k in range(KSUB):
            for j in range(SUB // 16):
                v = idx_v.at[b][k, pl.ds(j * 16, 16)]
                rel = v - rel_base
                ok = (rel >= 0) & (rel < HALF)
                idx_v.at[b][k, pl.ds(j * 16, 16)] = jnp.where(
                    ok, rel, jnp.full((16,), HALF, jnp.int32))

    def scatter(b):
        descs = [
            pltpu.async_copy(rows_v.at[b].at[pl.ds(k * SUB, SUB)],
                             acc.at[idx_v.at[b].at[k]], ssem, add=True)
            for k in range(KSUB)
        ]
        for d in descs:
            d.wait()

    for b in range(NBUF):
        @pl.when(b < trips)
        def _(b=b):
            fill_start(b, b)

    outer = (trips + NBUF - 1) // NBUF

    @pl.loop(0, outer)
    def _(g):
        for b in range(NBUF):
            t = g * NBUF + b

            @pl.when(t < trips)
            def _(t=t, b=b):
                fill_wait(t, b)
                clip(b)
                scatter(b)

            @pl.when(t + NBUF < trips)
            def _(t=t, b=b):
                fill_start(t + NBUF, b)

    plsc.subcore_barrier()

    for cond, obase, pieces in _owned((ROWS_A, ROWS_LAST)):
        @pl.when(cond)
        def _(obase=obase, pieces=pieces):
            off = 0
            for n in pieces:
                pltpu.sync_copy(
                    acc.at[pl.ds(obase + off, n)],
                    out_hbm.at[pl.ds(cid * HALF + obase + off, n)])
                off += n


@jax.jit
def kernel(src, indices):
    idx32 = indices.astype(jnp.int32)
    p = jnp.sum((idx32 < HALF).astype(jnp.int32))
    lo_end = (p + CHUNK - 1) // CHUNK
    hi_start = p // CHUNK
    lane0 = (jnp.arange(16, dtype=jnp.int32) == 0).astype(jnp.float32)
    bounds = jnp.stack([lane0 * lo_end.astype(jnp.float32),
                        lane0 * hi_start.astype(jnp.float32)])
    return _scatter_add(src, idx32.reshape(NCH_TOT, KSUB, SUB), bounds)


# speedup vs baseline: 1.1011x; 1.1011x over previous
import functools

import jax
import jax.numpy as jnp
from jax import lax
from jax.experimental import pallas as pl
from jax.experimental.pallas import tpu as pltpu
from jax.experimental.pallas import tpu_sc as plsc

NUM_NODES = 10000
NUM_EDGES = 320000
FEAT = 128

NC = 2
NS = 16
HALF = NUM_NODES // NC
ACC_ROWS = HALF + 8

CHUNK = 160
NCH_TOT = NUM_EDGES // CHUNK
SUB = 80
KSUB = CHUNK // SUB
NBUF = 4

ROWS_A = 312
ROWS_LAST = HALF - ROWS_A * (NS - 1)
ZROWS_A = 312
ZROWS_LAST = ACC_ROWS - ZROWS_A * (NS - 1)

_mesh = plsc.VectorSubcoreMesh(core_axis_name="c", subcore_axis_name="s")


def _split(count, piece=None):
    piece = CHUNK if piece is None else piece
    out = [piece] * (count // piece)
    if count % piece:
        out.append(count % piece)
    return out


@functools.partial(
    pl.kernel,
    out_type=jax.ShapeDtypeStruct((NUM_NODES, FEAT), jnp.float32),
    mesh=_mesh,
    scratch_types=[
        pltpu.VMEM_SHARED((ACC_ROWS, FEAT), jnp.float32),
        pltpu.VMEM((NBUF, CHUNK, FEAT), jnp.float32),
        pltpu.VMEM((NBUF, KSUB, SUB), jnp.int32),
        pltpu.VMEM((NC, 16), jnp.float32),
        pltpu.VMEM((40, FEAT), jnp.float32),
    ] + [pltpu.SemaphoreType.DMA] * (2 * NBUF + 1),
    compiler_params=pltpu.CompilerParams(needs_layout_passes=False),
)
def _scatter_add(src_hbm, idx_hbm, bounds_hbm, out_hbm, acc, rows_v, idx_v,
                 bounds_v, zero_v, *sems):
    rsems = sems[:NBUF]
    isems = sems[NBUF:2 * NBUF]
    ssem = sems[2 * NBUF]

    cid = lax.axis_index("c")
    sid = lax.axis_index("s")

    pltpu.sync_copy(bounds_hbm, bounds_v)
    lo_end = jnp.sum(bounds_v[0, :]).astype(jnp.int32)
    hi_start = jnp.sum(bounds_v[1, :]).astype(jnp.int32)
    start_c = jnp.where(cid == 0, 0, hi_start)
    end_c = jnp.where(cid == 0, lo_end, NCH_TOT)
    trips = jnp.maximum(0, (end_c - start_c - sid + NS - 1) // NS)

    def chunk_of(t):
        return start_c + sid + t * NS

    def fill_start(t, b):
        ch = chunk_of(t)
        pltpu.async_copy(src_hbm.at[pl.ds(ch * CHUNK, CHUNK)],
                         rows_v.at[b], rsems[b])
        pltpu.async_copy(idx_hbm.at[ch], idx_v.at[b], isems[b])

    def fill_wait(t, b):
        ch = chunk_of(t)
        pltpu.make_async_copy(src_hbm.at[pl.ds(ch * CHUNK, CHUNK)],
                              rows_v.at[b], rsems[b]).wait()
        pltpu.make_async_copy(idx_hbm.at[ch], idx_v.at[b],
                              isems[b]).wait()

    for b in range(NBUF):
        @pl.when(b < trips)
        def _(b=b):
            fill_start(b, b)

    @pl.loop(0, 40 * FEAT // 16)
    def _(i):
        r = i // (FEAT // 16)
        col = i % (FEAT // 16)
        zero_v[r, pl.ds(col * 16, 16)] = jnp.zeros((16,), jnp.float32)

    def _owned(plan, piece=None):
        a, last = plan
        yield (sid < NS - 1, sid * a, _split(a, piece))
        yield (sid == NS - 1, (NS - 1) * a, _split(last, piece))

    for cond, zbase, pieces in _owned((ZROWS_A, ZROWS_LAST), 40):
        @pl.when(cond)
        def _(zbase=zbase, pieces=pieces):
            off = 0
            for n in pieces:
                pltpu.sync_copy(zero_v.at[pl.ds(0, n)],
                                acc.at[pl.ds(zbase + off, n)])
                off += n
    plsc.subcore_barrier()

    rel_base = cid * HALF

    def clip(b):
        for k in range(KSUB):
            for j in range(SUB // 16):
                v = idx_v.at[b][k, pl.ds(j * 16, 16)]
                rel = v - rel_base
                ok = (rel >= 0) & (rel < HALF)
                idx_v.at[b][k, pl.ds(j * 16, 16)] = jnp.where(
                    ok, rel, jnp.full((16,), HALF, jnp.int32))

    def scatter(b):
        descs = [
            pltpu.async_copy(rows_v.at[b].at[pl.ds(k * SUB, SUB)],
                             acc.at[idx_v.at[b].at[k]], ssem, add=True)
            for k in range(KSUB)
        ]
        for d in descs:
            d.wait()

    outer = (trips + NBUF - 1) // NBUF

    @pl.loop(0, outer)
    def _(g):
        for b in range(NBUF):
            t = g * NBUF + b

            @pl.when(t < trips)
            def _(t=t, b=b):
                fill_wait(t, b)
                clip(b)
                scatter(b)

            @pl.when(t + NBUF < trips)
            def _(t=t, b=b):
                fill_start(t + NBUF, b)

    plsc.subcore_barrier()

    for cond, obase, pieces in _owned((ROWS_A, ROWS_LAST)):
        @pl.when(cond)
        def _(obase=obase, pieces=pieces):
            off = 0
            for n in pieces:
                pltpu.sync_copy(
                    acc.at[pl.ds(obase + off, n)],
                    out_hbm.at[pl.ds(cid * HALF + obase + off, n)])
                off += n


@jax.jit
def kernel(src, indices):
    idx32 = indices.astype(jnp.int32)
    p = jnp.sum((idx32 < HALF).astype(jnp.int32))
    lo_end = (p + CHUNK - 1) // CHUNK
    hi_start = p // CHUNK
    lane0 = (jnp.arange(16, dtype=jnp.int32) == 0).astype(jnp.float32)
    bounds = jnp.stack([lane0 * lo_end.astype(jnp.float32),
                        lane0 * hi_start.astype(jnp.float32)])
    return _scatter_add(src, idx32.reshape(NCH_TOT, KSUB, SUB), bounds)
